# baseline (speedup 1.0000x reference)
import functools

import jax
import jax.numpy as jnp
from jax import lax
from jax.experimental import pallas as pl
from jax.experimental.pallas import tpu as pltpu
from jax.experimental.pallas import tpu_sc as plsc

N = 10000
E = 320000
F_IN = 128
H = 128
C = 10
G = 64

NC = 2
NS = 16
NW = NC * NS
EW = E // NW
K = 80
CH = EW // K
NSLOT = 4
NP = 10112
ROWS_W = NP // NS

_HIGH = lax.Precision.HIGHEST


def _dot(a, b):
    return lax.dot_general(a, b, (((1,), (0,)), ((), ())),
                           precision=_HIGH, preferred_element_type=jnp.float32)


def _sc_mesh():
    return plsc.VectorSubcoreMesh(core_axis_name="c", subcore_axis_name="s")


@jax.jit
def _sc_degree(dst3, ones16, zeros16):
    @functools.partial(
        pl.kernel,
        out_type=jax.ShapeDtypeStruct((NC, NP, H), jnp.float32),
        mesh=_sc_mesh(),
        scratch_types=[
            pltpu.VMEM_SHARED((NP, H), jnp.float32),
            pltpu.VMEM((CH, K), jnp.int32),
            pltpu.VMEM((K, H), jnp.float32),
        ],
    )
    def deg_kernel(dst_hbm, ones_hbm, zeros_hbm, out_hbm, acc, didx, ones_v):
        cid = lax.axis_index("c")
        sid = lax.axis_index("s")
        wid = sid * NC + cid

        pltpu.sync_copy(zeros_hbm.at[pl.ds(sid * ROWS_W, ROWS_W)],
                        acc.at[pl.ds(sid * ROWS_W, ROWS_W)])
        pltpu.sync_copy(dst_hbm.at[wid], didx)
        pltpu.sync_copy(ones_hbm, ones_v)
        plsc.subcore_barrier()

        @pl.loop(0, CH)
        def _(c):
            pltpu.sync_copy(ones_v, acc.at[didx.at[c]], add=True)

        plsc.subcore_barrier()
        pltpu.sync_copy(acc.at[pl.ds(sid * ROWS_W, ROWS_W)],
                        out_hbm.at[cid, pl.ds(sid * ROWS_W, ROWS_W)])

    return deg_kernel(dst3, ones16, zeros16)


@jax.jit
def _sc_aggregate(hp, edge_t, zeros):
    @functools.partial(
        pl.kernel,
        out_type=jax.ShapeDtypeStruct((NC, NP, H), jnp.float32),
        mesh=_sc_mesh(),
        scratch_types=[
            pltpu.VMEM_SHARED((NP, H), jnp.float32),
            pltpu.VMEM((2, K), jnp.int32),
            pltpu.VMEM((2, K), jnp.int32),
            pltpu.VMEM((2, K), jnp.int32),
            pltpu.VMEM((2, K), jnp.int32),
            pltpu.VMEM((K, H), jnp.float32),
            pltpu.VMEM((K, H), jnp.float32),
            pltpu.VMEM((K, H), jnp.float32),
            pltpu.VMEM((K, H), jnp.float32),
            pltpu.SemaphoreType.DMA,
            pltpu.SemaphoreType.DMA,
            pltpu.SemaphoreType.DMA,
            pltpu.SemaphoreType.DMA,
            pltpu.SemaphoreType.DMA,
            pltpu.SemaphoreType.DMA,
            pltpu.SemaphoreType.DMA,
            pltpu.SemaphoreType.DMA,
            pltpu.SemaphoreType.DMA,
            pltpu.SemaphoreType.DMA,
            pltpu.SemaphoreType.DMA,
            pltpu.SemaphoreType.DMA,
        ],
    )
    def agg_kernel(hp_hbm, edge_hbm, zeros_hbm, out_hbm, acc,
                   i0, i1, i2, i3, b0, b1, b2, b3,
                   sI0, sI1, sI2, sI3, sG0, sG1, sG2, sG3,
                   sS0, sS1, sS2, sS3):
        cid = lax.axis_index("c")
        sid = lax.axis_index("s")
        wid = sid * NC + cid
        ibufs = (i0, i1, i2, i3)
        bufs = (b0, b1, b2, b3)
        sI = (sI0, sI1, sI2, sI3)
        sG = (sG0, sG1, sG2, sG3)
        sS = (sS0, sS1, sS2, sS3)

        def idx_fire(c, j):
            pltpu.async_copy(edge_hbm.at[wid, c], ibufs[j], sI[j])

        def idx_wait(c, j):
            pltpu.make_async_copy(edge_hbm.at[wid, c], ibufs[j], sI[j]).wait()

        def g_fire(j):
            pltpu.async_copy(hp_hbm.at[ibufs[j].at[0]], bufs[j], sG[j])

        def g_wait(j):
            pltpu.make_async_copy(
                hp_hbm.at[ibufs[j].at[0]], bufs[j], sG[j]).wait()

        def s_fire(j):
            pltpu.async_copy(bufs[j], acc.at[ibufs[j].at[1]], sS[j], add=True)

        def s_wait(j):
            pltpu.make_async_copy(
                bufs[j], acc.at[ibufs[j].at[1]], sS[j]).wait()

        pltpu.sync_copy(zeros_hbm.at[pl.ds(sid * ROWS_W, ROWS_W)],
                        acc.at[pl.ds(sid * ROWS_W, ROWS_W)])
        for j in range(NSLOT):
            idx_fire(j, j)
        plsc.subcore_barrier()
        for j in range(NSLOT):
            idx_wait(j, j)
            g_fire(j)

        @pl.loop(0, CH - 1, step=NSLOT)
        def _(c0):
            for j in range(NSLOT):
                g_wait(j)
                s_fire(j)
            for j in range(NSLOT):
                c = c0 + j

                @pl.when(c + NSLOT <= CH - 1)
                def _(c=c, j=j):
                    s_wait(j)
                    idx_fire(c + NSLOT, j)
            for j in range(NSLOT):
                c = c0 + j

                @pl.when(c + NSLOT <= CH - 1)
                def _(c=c, j=j):
                    idx_wait(c + NSLOT, j)
                    g_fire(j)

        g_wait(0)
        s_fire(0)
        for j in range(NSLOT):
            s_wait(j)

        plsc.subcore_barrier()
        pltpu.sync_copy(acc.at[pl.ds(sid * ROWS_W, ROWS_W)],
                        out_hbm.at[cid, pl.ds(sid * ROWS_W, ROWS_W)])

    return agg_kernel(hp, edge_t, zeros)


_R = 1000


def _tc_matmul(x, W):
    def body(x_ref, w_ref, o_ref):
        o_ref[...] = _dot(x_ref[...], w_ref[...])

    return pl.pallas_call(
        body,
        grid=(N // _R,),
        in_specs=[pl.BlockSpec((_R, F_IN), lambda i: (i, 0)),
                  pl.BlockSpec((F_IN, H), lambda i: (0, 0))],
        out_specs=pl.BlockSpec((_R, H), lambda i: (i, 0)),
        out_shape=jax.ShapeDtypeStruct((N, H), jnp.float32),
    )(x, W)


def _tc_prescale(t1, degp):
    def body(t_ref, d_ref, hp_ref, dis_ref):
        deg = d_ref[0, :, 0:1] + d_ref[1, :, 0:1] + 1.0
        dis = lax.rsqrt(deg)
        dis_ref[...] = dis
        hp_ref[...] = t_ref[...] * dis

    return pl.pallas_call(
        body,
        grid=(N // _R,),
        in_specs=[pl.BlockSpec((_R, H), lambda i: (i, 0)),
                  pl.BlockSpec((NC, _R, H), lambda i: (0, i, 0))],
        out_specs=[pl.BlockSpec((_R, H), lambda i: (i, 0)),
                   pl.BlockSpec((_R, 1), lambda i: (i, 0))],
        out_shape=[jax.ShapeDtypeStruct((N, H), jnp.float32),
                   jax.ShapeDtypeStruct((N, 1), jnp.float32)],
    )(t1, degp)


def _tc_layer(parts, hp, dis, b, W, relu=True):
    def body(p_ref, hp_ref, dis_ref, b_ref, w_ref, o_ref):
        dis = dis_ref[...]
        z = dis * (p_ref[0] + p_ref[1] + hp_ref[...]) + b_ref[...]
        if relu:
            z = jnp.maximum(z, 0.0)
        o_ref[...] = _dot(z, w_ref[...]) * dis

    return pl.pallas_call(
        body,
        grid=(N // _R,),
        in_specs=[pl.BlockSpec((NC, _R, H), lambda i: (0, i, 0)),
                  pl.BlockSpec((_R, H), lambda i: (i, 0)),
                  pl.BlockSpec((_R, 1), lambda i: (i, 0)),
                  pl.BlockSpec((1, H), lambda i: (0, 0)),
                  pl.BlockSpec((H, H), lambda i: (0, 0))],
        out_specs=pl.BlockSpec((_R, H), lambda i: (i, 0)),
        out_shape=jax.ShapeDtypeStruct((N, H), jnp.float32),
    )(parts, hp, dis, b, W)


def _tc_pool_head(parts, hp, dis, b3, batch2, Wl, bl):
    def body(p_ref, hp_ref, dis_ref, b_ref, bat_ref, wl_ref, bl_ref, o_ref):
        z = dis_ref[...] * (p_ref[0] + p_ref[1] + hp_ref[...]) + b_ref[...]
        gids = lax.broadcasted_iota(jnp.int32, (G, N), 0)
        mask = (bat_ref[...] == gids).astype(jnp.float32)
        sums = _dot(mask, z)
        counts = jnp.sum(mask, axis=1, keepdims=True)
        pooled = sums / jnp.maximum(counts, 1.0)
        o_ref[...] = _dot(pooled, wl_ref[...]) + bl_ref[...]

    return pl.pallas_call(
        body,
        grid=(1,),
        in_specs=[pl.BlockSpec((NC, N, H), lambda i: (0, 0, 0)),
                  pl.BlockSpec((N, H), lambda i: (0, 0)),
                  pl.BlockSpec((N, 1), lambda i: (0, 0)),
                  pl.BlockSpec((1, H), lambda i: (0, 0)),
                  pl.BlockSpec((1, N), lambda i: (0, 0)),
                  pl.BlockSpec((H, C), lambda i: (0, 0)),
                  pl.BlockSpec((1, C), lambda i: (0, 0))],
        out_specs=pl.BlockSpec((G, C), lambda i: (0, 0)),
        out_shape=jax.ShapeDtypeStruct((G, C), jnp.float32),
    )(parts, hp, dis, b3, batch2, Wl, bl)


def kernel(x, edge_index, batch, W1, b1, W2, b2, W3, b3, Wl, bl):
    dst3 = edge_index[1].reshape(NW, CH, K)
    edge_t = edge_index.reshape(2, NW, CH, K).transpose(1, 2, 0, 3)
    zeros = jnp.zeros((NP, H), jnp.float32)
    ones16 = jnp.ones((K, H), jnp.float32)
    batch2 = batch.reshape(1, N)
    b1r, b2r, b3r = b1.reshape(1, H), b2.reshape(1, H), b3.reshape(1, H)
    blr = bl.reshape(1, C)

    degp = _sc_degree(dst3, ones16, zeros)
    t1 = _tc_matmul(x, W1)
    h1p, dis = _tc_prescale(t1, degp)

    p1 = _sc_aggregate(h1p, edge_t, zeros)
    h2p = _tc_layer(p1, h1p, dis, b1r, W2, relu=True)

    p2 = _sc_aggregate(h2p, edge_t, zeros)
    h3p = _tc_layer(p2, h2p, dis, b2r, W3, relu=True)

    p3 = _sc_aggregate(h3p, edge_t, zeros)
    return _tc_pool_head(p3, h3p, dis, b3r, batch2, Wl, blr)
